# baseline (speedup 1.0000x reference)
import functools
import jax
import jax.numpy as jnp
from jax import lax
from jax.experimental import pallas as pl
from jax.experimental.pallas import tpu as pltpu
from jax.experimental.pallas import tpu_sc as plsc

N = 10000
E = 320000
NPAD = 10016
NACC = 10240
EPAD = 327680
CHUNK = 128
ROWS_PER_TILE = 640
TAB_PER_TILE = 626
PAD_SRC = N
PAD_DST = N
NBUF = 4
EC_CHUNKS = 80
CC_CHUNKS = 160
PHASES = 4
PH_CHUNKS = CC_CHUNKS // PHASES

_mesh = plsc.VectorSubcoreMesh(core_axis_name="c", subcore_axis_name="s")


@functools.partial(
    pl.kernel,
    out_type=jax.ShapeDtypeStruct((2 * NACC, 64), jnp.float32),
    mesh=_mesh,
    compiler_params=pltpu.CompilerParams(use_tc_tiling_on_sc=False),
    scratch_types=[
        pltpu.VMEM((PH_CHUNKS, CHUNK), jnp.int32),
        pltpu.VMEM((PH_CHUNKS, CHUNK), jnp.int32),
        pltpu.VMEM((PH_CHUNKS, CHUNK), jnp.int32),
        [pltpu.VMEM((CHUNK, 64), jnp.float32)
         for _ in range(NBUF)],
        pltpu.VMEM_SHARED((NACC, 64), jnp.float32),
        pltpu.VMEM_SHARED((NPAD, 64), jnp.float32),
        [pltpu.SemaphoreType.DMA for _ in range(NBUF)],
        [pltpu.SemaphoreType.DMA for _ in range(NBUF)],
    ],
)
def _sc_agg_chan(src_hbm, src2_hbm, dst_hbm, table_hbm, zrows_hbm, out_hbm,
                 src_v, src2_v, dst_v, rows_v, acc, stab, gsem, ssem):
    cid = lax.axis_index("c")
    sid = lax.axis_index("s")

    pltpu.sync_copy(zrows_hbm, acc.at[pl.ds(sid * ROWS_PER_TILE,
                                            ROWS_PER_TILE)])
    t0 = sid * TAB_PER_TILE
    pltpu.sync_copy(table_hbm.at[pl.ds(cid * NPAD + t0, TAB_PER_TILE)],
                    stab.at[pl.ds(t0, TAB_PER_TILE)])
    plsc.subcore_barrier()

    for ph in range(PHASES):
        base = sid * CC_CHUNKS + ph * PH_CHUNKS
        pltpu.sync_copy(src_hbm.at[pl.ds(base, PH_CHUNKS)], src_v)
        pltpu.sync_copy(src2_hbm.at[cid].at[pl.ds(base, PH_CHUNKS)], src2_v)
        pltpu.sync_copy(dst_hbm.at[pl.ds(base, PH_CHUNKS)], dst_v)

        def gat_hbm(row, b):
            pltpu.async_copy(table_hbm.at[src2_v.at[row]], rows_v[b],
                             gsem[b])

        def gat_spm(row, b):
            pltpu.async_copy(stab.at[src_v.at[row]], rows_v[b], gsem[b])

        gat_hbm(0, 0)
        for b in range(1, NBUF):
            gat_spm(b, b)

        def body(j, carry):
            for b in range(NBUF):
                jj = j * NBUF + b
                pltpu.make_async_copy(table_hbm.at[src_v.at[0]],
                                      rows_v[b], gsem[b]).wait()
                pltpu.async_copy(rows_v[b], acc.at[dst_v.at[jj]], ssem[b],
                                 add=True)
                b2 = (b - 2) % NBUF

                @pl.when(jj >= 2)
                def _():
                    pltpu.make_async_copy(rows_v[b2],
                                          acc.at[dst_v.at[0]],
                                          ssem[b2]).wait()

                can_refill = jnp.logical_and(jj >= 2, jj <= PH_CHUNKS - 3)

                if b2 == 0:
                    @pl.when(can_refill)
                    def _():
                        gat_hbm(jj + 2, 0)
                elif b2 == 1:
                    @pl.when(jnp.logical_and(can_refill, (j + 1) % 2 == 1))
                    def _():
                        gat_hbm(jj + 2, 1)

                    @pl.when(jnp.logical_and(can_refill, (j + 1) % 2 == 0))
                    def _():
                        gat_spm(jj + 2, 1)
                else:
                    @pl.when(can_refill)
                    def _():
                        gat_spm(jj + 2, b2)
            return carry

        lax.fori_loop(0, PH_CHUNKS // NBUF, body, 0)
        for b in ((PH_CHUNKS - 2) % NBUF, (PH_CHUNKS - 1) % NBUF):
            pltpu.make_async_copy(rows_v[b], acc.at[dst_v.at[0]],
                                  ssem[b]).wait()

    plsc.subcore_barrier()
    r0 = sid * ROWS_PER_TILE
    pltpu.sync_copy(acc.at[pl.ds(r0, ROWS_PER_TILE)],
                    out_hbm.at[pl.ds(cid * NACC + r0, ROWS_PER_TILE)])


@functools.partial(
    pl.kernel,
    out_type=jax.ShapeDtypeStruct((2 * NACC, 16), jnp.float32),
    mesh=_mesh,
    compiler_params=pltpu.CompilerParams(use_tc_tiling_on_sc=False),
    scratch_types=[
        pltpu.VMEM((EC_CHUNKS, CHUNK), jnp.int32),
        pltpu.VMEM((EC_CHUNKS, CHUNK), jnp.int32),
        [pltpu.VMEM((CHUNK, 16), jnp.float32)
         for _ in range(NBUF)],
        pltpu.VMEM_SHARED((NACC, 16), jnp.float32),
        pltpu.VMEM_SHARED((NPAD, 16), jnp.float32),
        [pltpu.SemaphoreType.DMA for _ in range(NBUF)],
        [pltpu.SemaphoreType.DMA for _ in range(NBUF)],
    ],
)
def _sc_agg16(src_hbm, dst_hbm, table_hbm, zrows_hbm, out_hbm,
              src_v, dst_v, rows_v, acc, stab, gsem, ssem):
    cid = lax.axis_index("c")
    sid = lax.axis_index("s")
    wid = cid * 16 + sid

    pltpu.sync_copy(zrows_hbm, acc.at[pl.ds(sid * ROWS_PER_TILE,
                                            ROWS_PER_TILE)])
    t0 = sid * TAB_PER_TILE
    pltpu.sync_copy(table_hbm.at[pl.ds(t0, TAB_PER_TILE)],
                    stab.at[pl.ds(t0, TAB_PER_TILE)])
    base = wid * EC_CHUNKS
    pltpu.sync_copy(src_hbm.at[pl.ds(base, EC_CHUNKS)], src_v)
    pltpu.sync_copy(dst_hbm.at[pl.ds(base, EC_CHUNKS)], dst_v)
    plsc.subcore_barrier()

    for b in range(NBUF):
        pltpu.async_copy(stab.at[src_v.at[b]], rows_v[b], gsem[b])

    def body(j, carry):
        for b in range(NBUF):
            jj = j * NBUF + b
            pltpu.make_async_copy(table_hbm.at[src_v.at[0]],
                                  rows_v[b], gsem[b]).wait()
            pltpu.async_copy(rows_v[b], acc.at[dst_v.at[jj]], ssem[b],
                             add=True)
            b2 = (b - 2) % NBUF

            @pl.when(jj >= 2)
            def _():
                pltpu.make_async_copy(rows_v[b2], acc.at[dst_v.at[0]],
                                      ssem[b2]).wait()

            @pl.when(jnp.logical_and(jj >= 2, jj <= EC_CHUNKS - 3))
            def _():
                pltpu.async_copy(stab.at[src_v.at[jj + 2]],
                                 rows_v[b2], gsem[b2])
        return carry

    lax.fori_loop(0, EC_CHUNKS // NBUF, body, 0)
    for b in ((EC_CHUNKS - 2) % NBUF, (EC_CHUNKS - 1) % NBUF):
        pltpu.make_async_copy(rows_v[b], acc.at[dst_v.at[0]],
                              ssem[b]).wait()
    plsc.subcore_barrier()

    r0 = sid * ROWS_PER_TILE
    pltpu.sync_copy(acc.at[pl.ds(r0, ROWS_PER_TILE)],
                    out_hbm.at[pl.ds(cid * NACC + r0, ROWS_PER_TILE)])


@functools.partial(
    pl.kernel,
    out_type=jax.ShapeDtypeStruct((2 * NACC, 16), jnp.float32),
    mesh=_mesh,
    compiler_params=pltpu.CompilerParams(use_tc_tiling_on_sc=False),
    scratch_types=[
        pltpu.VMEM((EC_CHUNKS, CHUNK), jnp.int32),
        pltpu.VMEM((CHUNK, 16), jnp.float32),
        pltpu.VMEM_SHARED((NACC, 16), jnp.float32),
        pltpu.SemaphoreType.DMA,
    ],
)
def _sc_deg(dst_hbm, ones_hbm, zrows_hbm, out_hbm, dst_v, ones_v, acc, sem):
    cid = lax.axis_index("c")
    sid = lax.axis_index("s")
    wid = cid * 16 + sid

    pltpu.sync_copy(zrows_hbm, acc.at[pl.ds(sid * ROWS_PER_TILE,
                                            ROWS_PER_TILE)])
    pltpu.sync_copy(dst_hbm.at[pl.ds(wid * EC_CHUNKS, EC_CHUNKS)], dst_v)
    pltpu.sync_copy(ones_hbm, ones_v)
    plsc.subcore_barrier()

    def body(j, carry):
        pltpu.async_copy(ones_v, acc.at[dst_v.at[j]], sem, add=True)
        return carry

    lax.fori_loop(0, EC_CHUNKS, body, 0)

    def drain(j, carry):
        pltpu.make_async_copy(ones_v, acc.at[dst_v.at[0]], sem).wait()
        return carry

    lax.fori_loop(0, EC_CHUNKS, drain, 0)
    plsc.subcore_barrier()

    r0 = sid * ROWS_PER_TILE
    pltpu.sync_copy(acc.at[pl.ds(r0, ROWS_PER_TILE)],
                    out_hbm.at[pl.ds(cid * NACC + r0, ROWS_PER_TILE)])


def _tc_layer1(x, W1, degout):
    def body(x_ref, w_ref, deg_ref, g_ref, dinv_ref):
        deg = (deg_ref[0:N, 0:1] + deg_ref[NACC:NACC + N, 0:1] + 1.0)
        dinv = lax.rsqrt(deg)
        h = jnp.dot(x_ref[...], w_ref[...],
                    preferred_element_type=jnp.float32)
        g = h * dinv
        g_ref[0:N, :] = g[:, 0:64]
        g_ref[N:NPAD, :] = jnp.zeros((NPAD - N, 64), jnp.float32)
        g_ref[NPAD:NPAD + N, :] = g[:, 64:128]
        g_ref[NPAD + N:, :] = jnp.zeros((NPAD - N, 64), jnp.float32)
        dinv_ref[0:N, :] = dinv
        dinv_ref[N:, :] = jnp.ones((NPAD - N, 1), jnp.float32)

    return pl.pallas_call(
        body,
        out_shape=(
            jax.ShapeDtypeStruct((2 * NPAD, 64), jnp.float32),
            jax.ShapeDtypeStruct((NPAD, 1), jnp.float32),
        ),
    )(x, W1, degout)


def _tc_layer2(agg1, gtab, dinv, b1, W2p):
    def body(a_ref, g_ref, dinv_ref, b1_ref, w2_ref, g2_ref):
        dinv = dinv_ref[...]
        a = jnp.concatenate([a_ref[0:NPAD, :], a_ref[NACC:NACC + NPAD, :]],
                            axis=1)
        g = jnp.concatenate([g_ref[0:NPAD, :], g_ref[NPAD:2 * NPAD, :]],
                            axis=1)
        s = (a + g) * dinv + b1_ref[...]
        r = jnp.maximum(s, 0.0)
        p = jnp.dot(r, w2_ref[...], preferred_element_type=jnp.float32)
        row = lax.broadcasted_iota(jnp.int32, (NPAD, 1), 0)
        g2_ref[...] = jnp.where(row < N, p * dinv, 0.0)

    return pl.pallas_call(
        body,
        out_shape=jax.ShapeDtypeStruct((NPAD, 16), jnp.float32),
    )(agg1, gtab, dinv, b1, W2p)


def _tc_final(agg2, g2, dinv, b2p):
    def body(a_ref, g2_ref, dinv_ref, b2_ref, o_ref):
        o_ref[...] = ((a_ref[0:N, :] + a_ref[NACC:NACC + N, :]
                       + g2_ref[0:N, :]) * dinv_ref[0:N, :] + b2_ref[...])

    return pl.pallas_call(
        body,
        out_shape=jax.ShapeDtypeStruct((N, 16), jnp.float32),
    )(agg2, g2, dinv, b2p)


@jax.jit
def kernel(x, edge_index, W1, b1, W2, b2):
    src = edge_index[0].astype(jnp.int32)
    dst = edge_index[1].astype(jnp.int32)
    pad = jnp.full((EPAD - E,), PAD_SRC, jnp.int32)
    src2d = jnp.concatenate([src, pad]).reshape(EPAD // CHUNK, CHUNK)
    dst2d = jnp.concatenate([dst, jnp.full((EPAD - E,), PAD_DST, jnp.int32)]
                            ).reshape(EPAD // CHUNK, CHUNK)

    zrows64 = jnp.zeros((ROWS_PER_TILE, 64), jnp.float32)
    zrows16 = jnp.zeros((ROWS_PER_TILE, 16), jnp.float32)
    ones16 = jnp.ones((CHUNK, 16), jnp.float32)

    degout = _sc_deg(dst2d, ones16, zrows16)

    gtab, dinv = _tc_layer1(x, W1, degout)

    srcs2 = jnp.stack([src2d, src2d + NPAD])
    agg1 = _sc_agg_chan(src2d, srcs2, dst2d, gtab, zrows64)

    W2p = jnp.zeros((128, 16), jnp.float32).at[:, :3].set(W2)
    b1r = b1.reshape(1, 128)
    g2 = _tc_layer2(agg1, gtab, dinv, b1r, W2p)

    agg2 = _sc_agg16(src2d, dst2d, g2, zrows16)

    b2p = jnp.zeros((1, 16), jnp.float32).at[0, :3].set(b2)
    out = _tc_final(agg2, g2, dinv, b2p)
    return out[:, :3]


# speedup vs baseline: 1.2767x; 1.2767x over previous
import functools
import jax
import jax.numpy as jnp
from jax import lax
from jax.experimental import pallas as pl
from jax.experimental.pallas import tpu as pltpu
from jax.experimental.pallas import tpu_sc as plsc

N = 10000
E = 320000
NPAD = 10016
NACC = 10240
EPAD = 327680
CHUNK = 128
ROWS_PER_TILE = 640
TAB_PER_TILE = 626
PAD_SRC = N
PAD_DST = N
NBUF = 4
EC_CHUNKS = 80
CC_CHUNKS = 160
PHASES = 4
PH_CHUNKS = CC_CHUNKS // PHASES

_mesh = plsc.VectorSubcoreMesh(core_axis_name="c", subcore_axis_name="s")


@functools.partial(
    pl.kernel,
    out_type=jax.ShapeDtypeStruct((2 * NACC, 64), jnp.float32),
    mesh=_mesh,
    compiler_params=pltpu.CompilerParams(use_tc_tiling_on_sc=False),
    scratch_types=[
        pltpu.VMEM((PH_CHUNKS, CHUNK), jnp.int32),
        pltpu.VMEM((PH_CHUNKS, CHUNK), jnp.int32),
        [pltpu.VMEM((CHUNK, 64), jnp.float32)
         for _ in range(NBUF)],
        pltpu.VMEM_SHARED((NACC, 64), jnp.float32),
        pltpu.VMEM_SHARED((NPAD, 64), jnp.float32),
        [pltpu.SemaphoreType.DMA for _ in range(NBUF)],
        [pltpu.SemaphoreType.DMA for _ in range(NBUF)],
    ],
)
def _sc_agg_chan(src_hbm, dst_hbm, table_hbm, zrows_hbm, out_hbm,
                 src_v, dst_v, rows_v, acc, stab, gsem, ssem):
    cid = lax.axis_index("c")
    sid = lax.axis_index("s")

    pltpu.sync_copy(zrows_hbm, acc.at[pl.ds(sid * ROWS_PER_TILE,
                                            ROWS_PER_TILE)])
    t0 = sid * TAB_PER_TILE
    pltpu.sync_copy(table_hbm.at[pl.ds(cid * NPAD + t0, TAB_PER_TILE)],
                    stab.at[pl.ds(t0, TAB_PER_TILE)])
    plsc.subcore_barrier()

    for ph in range(PHASES):
        base = sid * CC_CHUNKS + ph * PH_CHUNKS
        pltpu.sync_copy(src_hbm.at[pl.ds(base, PH_CHUNKS)], src_v)
        pltpu.sync_copy(dst_hbm.at[pl.ds(base, PH_CHUNKS)], dst_v)

        def gat_spm(row, b):
            pltpu.async_copy(stab.at[src_v.at[row]], rows_v[b], gsem[b])

        for b in range(NBUF):
            gat_spm(b, b)

        def body(j, carry):
            for b in range(NBUF):
                jj = j * NBUF + b
                pltpu.make_async_copy(table_hbm.at[src_v.at[0]],
                                      rows_v[b], gsem[b]).wait()
                pltpu.async_copy(rows_v[b], acc.at[dst_v.at[jj]], ssem[b],
                                 add=True)
                b2 = (b - 2) % NBUF

                @pl.when(jj >= 2)
                def _():
                    pltpu.make_async_copy(rows_v[b2],
                                          acc.at[dst_v.at[0]],
                                          ssem[b2]).wait()

                can_refill = jnp.logical_and(jj >= 2, jj <= PH_CHUNKS - 3)

                @pl.when(can_refill)
                def _():
                    gat_spm(jj + 2, b2)
            return carry

        lax.fori_loop(0, PH_CHUNKS // NBUF, body, 0)
        for b in ((PH_CHUNKS - 2) % NBUF, (PH_CHUNKS - 1) % NBUF):
            pltpu.make_async_copy(rows_v[b], acc.at[dst_v.at[0]],
                                  ssem[b]).wait()

    plsc.subcore_barrier()
    r0 = sid * ROWS_PER_TILE
    pltpu.sync_copy(acc.at[pl.ds(r0, ROWS_PER_TILE)],
                    out_hbm.at[pl.ds(cid * NACC + r0, ROWS_PER_TILE)])


@functools.partial(
    pl.kernel,
    out_type=jax.ShapeDtypeStruct((2 * NACC, 16), jnp.float32),
    mesh=_mesh,
    compiler_params=pltpu.CompilerParams(use_tc_tiling_on_sc=False),
    scratch_types=[
        pltpu.VMEM((EC_CHUNKS, CHUNK), jnp.int32),
        pltpu.VMEM((EC_CHUNKS, CHUNK), jnp.int32),
        [pltpu.VMEM((CHUNK, 16), jnp.float32)
         for _ in range(NBUF)],
        pltpu.VMEM_SHARED((NACC, 16), jnp.float32),
        pltpu.VMEM_SHARED((NPAD, 16), jnp.float32),
        [pltpu.SemaphoreType.DMA for _ in range(NBUF)],
        [pltpu.SemaphoreType.DMA for _ in range(NBUF)],
    ],
)
def _sc_agg16(src_hbm, dst_hbm, table_hbm, zrows_hbm, out_hbm,
              src_v, dst_v, rows_v, acc, stab, gsem, ssem):
    cid = lax.axis_index("c")
    sid = lax.axis_index("s")
    wid = cid * 16 + sid

    pltpu.sync_copy(zrows_hbm, acc.at[pl.ds(sid * ROWS_PER_TILE,
                                            ROWS_PER_TILE)])
    t0 = sid * TAB_PER_TILE
    pltpu.sync_copy(table_hbm.at[pl.ds(t0, TAB_PER_TILE)],
                    stab.at[pl.ds(t0, TAB_PER_TILE)])
    base = wid * EC_CHUNKS
    pltpu.sync_copy(src_hbm.at[pl.ds(base, EC_CHUNKS)], src_v)
    pltpu.sync_copy(dst_hbm.at[pl.ds(base, EC_CHUNKS)], dst_v)
    plsc.subcore_barrier()

    for b in range(NBUF):
        pltpu.async_copy(stab.at[src_v.at[b]], rows_v[b], gsem[b])

    def body(j, carry):
        for b in range(NBUF):
            jj = j * NBUF + b
            pltpu.make_async_copy(table_hbm.at[src_v.at[0]],
                                  rows_v[b], gsem[b]).wait()
            pltpu.async_copy(rows_v[b], acc.at[dst_v.at[jj]], ssem[b],
                             add=True)
            b2 = (b - 2) % NBUF

            @pl.when(jj >= 2)
            def _():
                pltpu.make_async_copy(rows_v[b2], acc.at[dst_v.at[0]],
                                      ssem[b2]).wait()

            @pl.when(jnp.logical_and(jj >= 2, jj <= EC_CHUNKS - 3))
            def _():
                pltpu.async_copy(stab.at[src_v.at[jj + 2]],
                                 rows_v[b2], gsem[b2])
        return carry

    lax.fori_loop(0, EC_CHUNKS // NBUF, body, 0)
    for b in ((EC_CHUNKS - 2) % NBUF, (EC_CHUNKS - 1) % NBUF):
        pltpu.make_async_copy(rows_v[b], acc.at[dst_v.at[0]],
                              ssem[b]).wait()
    plsc.subcore_barrier()

    r0 = sid * ROWS_PER_TILE
    pltpu.sync_copy(acc.at[pl.ds(r0, ROWS_PER_TILE)],
                    out_hbm.at[pl.ds(cid * NACC + r0, ROWS_PER_TILE)])


@functools.partial(
    pl.kernel,
    out_type=jax.ShapeDtypeStruct((2 * NACC, 16), jnp.float32),
    mesh=_mesh,
    compiler_params=pltpu.CompilerParams(use_tc_tiling_on_sc=False),
    scratch_types=[
        pltpu.VMEM((EC_CHUNKS, CHUNK), jnp.int32),
        pltpu.VMEM((CHUNK, 16), jnp.float32),
        pltpu.VMEM_SHARED((NACC, 16), jnp.float32),
        pltpu.SemaphoreType.DMA,
    ],
)
def _sc_deg(dst_hbm, ones_hbm, zrows_hbm, out_hbm, dst_v, ones_v, acc, sem):
    cid = lax.axis_index("c")
    sid = lax.axis_index("s")
    wid = cid * 16 + sid

    pltpu.sync_copy(zrows_hbm, acc.at[pl.ds(sid * ROWS_PER_TILE,
                                            ROWS_PER_TILE)])
    pltpu.sync_copy(dst_hbm.at[pl.ds(wid * EC_CHUNKS, EC_CHUNKS)], dst_v)
    pltpu.sync_copy(ones_hbm, ones_v)
    plsc.subcore_barrier()

    def body(j, carry):
        pltpu.async_copy(ones_v, acc.at[dst_v.at[j]], sem, add=True)
        return carry

    lax.fori_loop(0, EC_CHUNKS, body, 0)

    def drain(j, carry):
        pltpu.make_async_copy(ones_v, acc.at[dst_v.at[0]], sem).wait()
        return carry

    lax.fori_loop(0, EC_CHUNKS, drain, 0)
    plsc.subcore_barrier()

    r0 = sid * ROWS_PER_TILE
    pltpu.sync_copy(acc.at[pl.ds(r0, ROWS_PER_TILE)],
                    out_hbm.at[pl.ds(cid * NACC + r0, ROWS_PER_TILE)])


def _tc_layer1(x, W1, degout):
    def body(x_ref, w_ref, deg_ref, g_ref, dinv_ref):
        deg = (deg_ref[0:N, 0:1] + deg_ref[NACC:NACC + N, 0:1] + 1.0)
        dinv = lax.rsqrt(deg)
        h = jnp.dot(x_ref[...], w_ref[...],
                    preferred_element_type=jnp.float32)
        g = h * dinv
        g_ref[0:N, :] = g[:, 0:64]
        g_ref[N:NPAD, :] = jnp.zeros((NPAD - N, 64), jnp.float32)
        g_ref[NPAD:NPAD + N, :] = g[:, 64:128]
        g_ref[NPAD + N:, :] = jnp.zeros((NPAD - N, 64), jnp.float32)
        dinv_ref[0:N, :] = dinv
        dinv_ref[N:, :] = jnp.ones((NPAD - N, 1), jnp.float32)

    return pl.pallas_call(
        body,
        out_shape=(
            jax.ShapeDtypeStruct((2 * NPAD, 64), jnp.float32),
            jax.ShapeDtypeStruct((NPAD, 1), jnp.float32),
        ),
    )(x, W1, degout)


def _tc_layer2(agg1, gtab, dinv, b1, W2p):
    def body(a_ref, g_ref, dinv_ref, b1_ref, w2_ref, g2_ref):
        dinv = dinv_ref[...]
        a = jnp.concatenate([a_ref[0:NPAD, :], a_ref[NACC:NACC + NPAD, :]],
                            axis=1)
        g = jnp.concatenate([g_ref[0:NPAD, :], g_ref[NPAD:2 * NPAD, :]],
                            axis=1)
        s = (a + g) * dinv + b1_ref[...]
        r = jnp.maximum(s, 0.0)
        p = jnp.dot(r, w2_ref[...], preferred_element_type=jnp.float32)
        row = lax.broadcasted_iota(jnp.int32, (NPAD, 1), 0)
        g2_ref[...] = jnp.where(row < N, p * dinv, 0.0)

    return pl.pallas_call(
        body,
        out_shape=jax.ShapeDtypeStruct((NPAD, 16), jnp.float32),
    )(agg1, gtab, dinv, b1, W2p)


def _tc_final(agg2, g2, dinv, b2p):
    def body(a_ref, g2_ref, dinv_ref, b2_ref, o_ref):
        o_ref[...] = ((a_ref[0:N, :] + a_ref[NACC:NACC + N, :]
                       + g2_ref[0:N, :]) * dinv_ref[0:N, :] + b2_ref[...])

    return pl.pallas_call(
        body,
        out_shape=jax.ShapeDtypeStruct((N, 16), jnp.float32),
    )(agg2, g2, dinv, b2p)


@jax.jit
def kernel(x, edge_index, W1, b1, W2, b2):
    src = edge_index[0].astype(jnp.int32)
    dst = edge_index[1].astype(jnp.int32)
    pad = jnp.full((EPAD - E,), PAD_SRC, jnp.int32)
    src2d = jnp.concatenate([src, pad]).reshape(EPAD // CHUNK, CHUNK)
    dst2d = jnp.concatenate([dst, jnp.full((EPAD - E,), PAD_DST, jnp.int32)]
                            ).reshape(EPAD // CHUNK, CHUNK)

    zrows64 = jnp.zeros((ROWS_PER_TILE, 64), jnp.float32)
    zrows16 = jnp.zeros((ROWS_PER_TILE, 16), jnp.float32)
    ones16 = jnp.ones((CHUNK, 16), jnp.float32)

    degout = _sc_deg(dst2d, ones16, zrows16)

    gtab, dinv = _tc_layer1(x, W1, degout)

    agg1 = _sc_agg_chan(src2d, dst2d, gtab, zrows64)

    W2p = jnp.zeros((128, 16), jnp.float32).at[:, :3].set(W2)
    b1r = b1.reshape(1, 128)
    g2 = _tc_layer2(agg1, gtab, dinv, b1r, W2p)

    agg2 = _sc_agg16(src2d, dst2d, g2, zrows16)

    b2p = jnp.zeros((1, 16), jnp.float32).at[0, :3].set(b2)
    out = _tc_final(agg2, g2, dinv, b2p)
    return out[:, :3]
